# baseline (speedup 1.0000x reference)
import functools

import numpy as np
import jax
import jax.numpy as jnp
from jax import lax
from jax.experimental import pallas as pl
from jax.experimental.pallas import tpu as pltpu
from jax.experimental.pallas import tpu_sc as plsc

N_NODES = 10000
N_EDGES = 320000
F = 128

NC = 2
NS = 16
NW = NC * NS

CHUNK = 128
NBUF = 2
K0 = 116
K1 = 42
E_PAD = NS * (K0 + K1) * CHUNK
E_SPLIT = NS * K0 * CHUNK
PAD_ROWS = 8

ROWS_MAIN = 624
LAST_START = (NS - 1) * ROWS_MAIN
LAST_ROWS = N_NODES - LAST_START

_sc_mesh = plsc.VectorSubcoreMesh(core_axis_name="c", subcore_axis_name="s")

_Q = np.empty((F,), np.int32)
for _g in range(F // 32):
    for _l in range(16):
        _Q[32 * _g + 2 * _l] = 32 * _g + _l
        _Q[32 * _g + 2 * _l + 1] = 32 * _g + 16 + _l


@functools.partial(
    pl.kernel,
    out_type=jax.ShapeDtypeStruct((NC, N_NODES, F), jnp.float32),
    mesh=_sc_mesh,
    scratch_types=[
        pltpu.VMEM_SHARED((N_NODES + PAD_ROWS, F), jnp.float32),
        pltpu.VMEM((NBUF, 2, CHUNK), jnp.int32),
        pltpu.VMEM((K1, 2, CHUNK), jnp.int32),
        pltpu.VMEM((NBUF, CHUNK, F // 2), jnp.int32),
        pltpu.VMEM((CHUNK, F), jnp.float32),
        [pltpu.SemaphoreType.DMA for _ in range(NBUF)],
    ],
    compiler_params=pltpu.CompilerParams(use_tc_tiling_on_sc=False),
)
def _sc_aggregate(xh_hbm, idx0_hbm, idx1_hbm, parts_hbm,
                  agg_s, idx_r, idx_all, rows_h, rows_f, sem_g):
    c = lax.axis_index("c")
    s = lax.axis_index("s")

    start = pl.multiple_of(s * ROWS_MAIN, 8)

    zv = jnp.zeros((16,), jnp.float32)

    def _zrow(r, carry):
        for kk in range(F // 16):
            rows_f[r, pl.ds(kk * 16, 16)] = zv
        return carry

    lax.fori_loop(0, CHUNK, _zrow, 0)

    @pl.when(s < NS - 1)
    def _():
        for i in range(4):
            pltpu.sync_copy(rows_f,
                            agg_s.at[pl.ds(start + i * CHUNK, CHUNK)])
        pltpu.sync_copy(rows_f.at[pl.ds(0, ROWS_MAIN - 4 * CHUNK)],
                        agg_s.at[pl.ds(start + 4 * CHUNK,
                                       ROWS_MAIN - 4 * CHUNK)])

    @pl.when(s == NS - 1)
    def _():
        for i in range(5):
            pltpu.sync_copy(rows_f,
                            agg_s.at[pl.ds(LAST_START + i * CHUNK, CHUNK)])
        pltpu.sync_copy(rows_f.at[pl.ds(0, PAD_ROWS)],
                        agg_s.at[pl.ds(LAST_START + 5 * CHUNK, PAD_ROWS)])

    plsc.subcore_barrier()

    def _widen(b):
        shift = jnp.full((16,), 16, jnp.int32)
        mask = jnp.full((16,), -65536, jnp.int32)

        def _rowgrp(r, carry):
            for g in range(F // 32):
                v = rows_h[b, r, pl.ds(g * 16, 16)]
                lo = lax.bitcast_convert_type(lax.shift_left(v, shift), jnp.float32)
                hi = lax.bitcast_convert_type(lax.bitwise_and(v, mask), jnp.float32)
                rows_f[r, pl.ds(g * 32, 16)] = lo
                rows_f[r, pl.ds(g * 32 + 16, 16)] = hi
            return carry

        lax.fori_loop(0, CHUNK, _rowgrp, 0)

    @pl.when(c == 0)
    def _():
        def _round(g, carry):
            base = g * NBUF
            pltpu.sync_copy(idx0_hbm.at[s, pl.ds(base, NBUF)], idx_r)
            dg = []
            for b in range(NBUF):
                dg.append(
                    pltpu.async_copy(xh_hbm.at[idx_r.at[b, 0]], rows_h.at[b],
                                     sem_g[b])
                )
            for b in range(NBUF):
                dg[b].wait()
                _widen(b)
                pltpu.sync_copy(rows_f, agg_s.at[idx_r.at[b, 1]],
                                add=True)
            return carry

        lax.fori_loop(0, K0 // NBUF, _round, 0)

    @pl.when(c == 1)
    def _():
        pltpu.sync_copy(idx1_hbm.at[s], idx_all)

        def _round(g, carry):
            base = g * NBUF
            dg = []
            for b in range(NBUF):
                j = base + b
                dg.append(
                    pltpu.async_copy(xh_hbm.at[idx_all.at[j, 0]], rows_h.at[b],
                                     sem_g[b])
                )
            for b in range(NBUF):
                j = base + b
                dg[b].wait()
                _widen(b)
                pltpu.sync_copy(rows_f, agg_s.at[idx_all.at[j, 1]],
                                add=True)
            return carry

        lax.fori_loop(0, K1 // NBUF, _round, 0)

    plsc.subcore_barrier()

    @pl.when(s < NS - 1)
    def _():
        pltpu.sync_copy(agg_s.at[pl.ds(start, ROWS_MAIN)],
                        parts_hbm.at[c, pl.ds(start, ROWS_MAIN)])

    @pl.when(s == NS - 1)
    def _():
        pltpu.sync_copy(agg_s.at[pl.ds(LAST_START, LAST_ROWS)],
                        parts_hbm.at[c, pl.ds(LAST_START, LAST_ROWS)])


def _tc_body(x_ref, p0_ref, p1_ref, wr_ref, wn_ref, b_ref, o_ref):
    agg = p0_ref[...] + p1_ref[...]
    acc = jnp.dot(x_ref[...], wr_ref[...], preferred_element_type=jnp.float32)
    acc = acc + jnp.dot(agg, wn_ref[...], preferred_element_type=jnp.float32)
    o_ref[...] = jnp.maximum(acc + b_ref[...], 0.0)


_ROW_BLK = 1000

_tc_finish = pl.pallas_call(
    _tc_body,
    grid=(N_NODES // _ROW_BLK,),
    in_specs=[
        pl.BlockSpec((_ROW_BLK, F), lambda i: (i, 0)),
        pl.BlockSpec((_ROW_BLK, F), lambda i: (i, 0)),
        pl.BlockSpec((_ROW_BLK, F), lambda i: (i, 0)),
        pl.BlockSpec((F, F), lambda i: (0, 0)),
        pl.BlockSpec((F, F), lambda i: (0, 0)),
        pl.BlockSpec((1, F), lambda i: (0, 0)),
    ],
    out_specs=pl.BlockSpec((_ROW_BLK, F), lambda i: (i, 0)),
    out_shape=jax.ShapeDtypeStruct((N_NODES, F), jnp.float32),
)


@jax.jit
def kernel(x, edge_index, W_root, W_nbr, b):
    ei = edge_index.astype(jnp.int32)
    pad = E_PAD - N_EDGES
    src = jnp.concatenate([ei[0], jnp.zeros((pad,), jnp.int32)])
    dst = jnp.concatenate([ei[1], jnp.full((pad,), N_NODES, jnp.int32)])
    idx0 = jnp.stack([src[:E_SPLIT].reshape(NS, K0, CHUNK),
                      dst[:E_SPLIT].reshape(NS, K0, CHUNK)], axis=2)
    idx1 = jnp.stack([src[E_SPLIT:].reshape(NS, K1, CHUNK),
                      dst[E_SPLIT:].reshape(NS, K1, CHUNK)], axis=2)
    xh = jnp.take(x, jnp.asarray(_Q), axis=1).astype(jnp.bfloat16)
    xh32 = lax.bitcast_convert_type(xh.reshape(N_NODES, F // 2, 2),
                                    jnp.int32)
    parts = _sc_aggregate(xh32, idx0, idx1)
    return _tc_finish(x, parts[0], parts[1], W_root, W_nbr,
                      b.reshape(1, F))


# speedup vs baseline: 1.8377x; 1.8377x over previous
import functools

import jax
import jax.numpy as jnp
from jax import lax
from jax.experimental import pallas as pl
from jax.experimental.pallas import tpu as pltpu
from jax.experimental.pallas import tpu_sc as plsc

N_NODES = 10000
N_EDGES = 320000
F = 128

NC = 2
NS = 16
NW = NC * NS

CHUNK = 128
NBUF = 2
K0 = 116
K1 = 42
E_PAD = NS * (K0 + K1) * CHUNK
E_SPLIT = NS * K0 * CHUNK
PAD_ROWS = 8

ROWS_MAIN = 624
LAST_START = (NS - 1) * ROWS_MAIN
LAST_ROWS = N_NODES - LAST_START

_sc_mesh = plsc.VectorSubcoreMesh(core_axis_name="c", subcore_axis_name="s")


@functools.partial(
    pl.kernel,
    out_type=jax.ShapeDtypeStruct((NC, N_NODES, F), jnp.float32),
    mesh=_sc_mesh,
    scratch_types=[
        pltpu.VMEM_SHARED((N_NODES + PAD_ROWS, F), jnp.float32),
        pltpu.VMEM((NBUF, 2, CHUNK), jnp.int32),
        pltpu.VMEM((K1, 2, CHUNK), jnp.int32),
        pltpu.VMEM((NBUF, CHUNK, F), jnp.float32),
        [pltpu.SemaphoreType.DMA for _ in range(NBUF)],
    ],
)
def _sc_aggregate(x_hbm, idx0_hbm, idx1_hbm, parts_hbm,
                  agg_s, idx_r, idx_all, rows, sem_g):
    c = lax.axis_index("c")
    s = lax.axis_index("s")

    start = pl.multiple_of(s * ROWS_MAIN, 8)

    zv = jnp.zeros((16,), jnp.float32)

    def _zrow(r, carry):
        for kk in range(F // 16):
            rows[0, r, pl.ds(kk * 16, 16)] = zv
        return carry

    lax.fori_loop(0, CHUNK, _zrow, 0)

    @pl.when(s < NS - 1)
    def _():
        for i in range(4):
            pltpu.sync_copy(rows.at[0],
                            agg_s.at[pl.ds(start + i * CHUNK, CHUNK)])
        pltpu.sync_copy(rows.at[0, pl.ds(0, ROWS_MAIN - 4 * CHUNK)],
                        agg_s.at[pl.ds(start + 4 * CHUNK,
                                       ROWS_MAIN - 4 * CHUNK)])

    @pl.when(s == NS - 1)
    def _():
        for i in range(5):
            pltpu.sync_copy(rows.at[0],
                            agg_s.at[pl.ds(LAST_START + i * CHUNK, CHUNK)])
        pltpu.sync_copy(rows.at[0, pl.ds(0, PAD_ROWS)],
                        agg_s.at[pl.ds(LAST_START + 5 * CHUNK, PAD_ROWS)])

    plsc.subcore_barrier()

    @pl.when(c == 0)
    def _():
        def _round(g, carry):
            base = g * NBUF
            pltpu.sync_copy(idx0_hbm.at[s, pl.ds(base, NBUF)], idx_r)
            dg = []
            for b in range(NBUF):
                dg.append(
                    pltpu.async_copy(x_hbm.at[idx_r.at[b, 0]], rows.at[b],
                                     sem_g[b])
                )
            for b in range(NBUF):
                dg[b].wait()
                pltpu.sync_copy(rows.at[b], agg_s.at[idx_r.at[b, 1]],
                                add=True)
            return carry

        lax.fori_loop(0, K0 // NBUF, _round, 0)

    @pl.when(c == 1)
    def _():
        pltpu.sync_copy(idx1_hbm.at[s], idx_all)

        def _round(g, carry):
            base = g * NBUF
            dg = []
            for b in range(NBUF):
                j = base + b
                dg.append(
                    pltpu.async_copy(x_hbm.at[idx_all.at[j, 0]], rows.at[b],
                                     sem_g[b])
                )
            for b in range(NBUF):
                j = base + b
                dg[b].wait()
                pltpu.sync_copy(rows.at[b], agg_s.at[idx_all.at[j, 1]],
                                add=True)
            return carry

        lax.fori_loop(0, K1 // NBUF, _round, 0)

    plsc.subcore_barrier()

    @pl.when(s < NS - 1)
    def _():
        pltpu.sync_copy(agg_s.at[pl.ds(start, ROWS_MAIN)],
                        parts_hbm.at[c, pl.ds(start, ROWS_MAIN)])

    @pl.when(s == NS - 1)
    def _():
        pltpu.sync_copy(agg_s.at[pl.ds(LAST_START, LAST_ROWS)],
                        parts_hbm.at[c, pl.ds(LAST_START, LAST_ROWS)])


def _tc_body(x_ref, p0_ref, p1_ref, wr_ref, wn_ref, b_ref, o_ref):
    agg = p0_ref[...] + p1_ref[...]
    acc = jnp.dot(x_ref[...], wr_ref[...], preferred_element_type=jnp.float32)
    acc = acc + jnp.dot(agg, wn_ref[...], preferred_element_type=jnp.float32)
    o_ref[...] = jnp.maximum(acc + b_ref[...], 0.0)


_ROW_BLK = 1000

_tc_finish = pl.pallas_call(
    _tc_body,
    grid=(N_NODES // _ROW_BLK,),
    in_specs=[
        pl.BlockSpec((_ROW_BLK, F), lambda i: (i, 0)),
        pl.BlockSpec((_ROW_BLK, F), lambda i: (i, 0)),
        pl.BlockSpec((_ROW_BLK, F), lambda i: (i, 0)),
        pl.BlockSpec((F, F), lambda i: (0, 0)),
        pl.BlockSpec((F, F), lambda i: (0, 0)),
        pl.BlockSpec((1, F), lambda i: (0, 0)),
    ],
    out_specs=pl.BlockSpec((_ROW_BLK, F), lambda i: (i, 0)),
    out_shape=jax.ShapeDtypeStruct((N_NODES, F), jnp.float32),
)


@jax.jit
def kernel(x, edge_index, W_root, W_nbr, b):
    ei = edge_index.astype(jnp.int32)
    pad = E_PAD - N_EDGES
    src = jnp.concatenate([ei[0], jnp.zeros((pad,), jnp.int32)])
    dst = jnp.concatenate([ei[1], jnp.full((pad,), N_NODES, jnp.int32)])
    idx0 = jnp.stack([src[:E_SPLIT].reshape(NS, K0, CHUNK),
                      dst[:E_SPLIT].reshape(NS, K0, CHUNK)], axis=2)
    idx1 = jnp.stack([src[E_SPLIT:].reshape(NS, K1, CHUNK),
                      dst[E_SPLIT:].reshape(NS, K1, CHUNK)], axis=2)
    parts = _sc_aggregate(x, idx0, idx1)
    return _tc_finish(x, parts[0], parts[1], W_root, W_nbr,
                      b.reshape(1, F))
